# baseline (speedup 1.0000x reference)
import functools

import jax
import jax.numpy as jnp
from jax import lax
from jax.experimental import pallas as pl
from jax.experimental.pallas import tpu as pltpu
from jax.experimental.pallas import tpu_sc as plsc

K_CODES = 8192
LATENT = 32
BETA = 0.25
TOK_BLK = 1024
CODE_BLK = 512
N_CODE_BLKS = K_CODES // CODE_BLK
N_TOKENS = 8192


def _dist_argmin_body(z_ref, e_ref, idx_ref, loss_ref, zq_ref):
    i = pl.program_id(0)
    z = z_ref[...]
    e = e_ref[...]
    sz = jnp.sum(z * z, axis=1, keepdims=True)

    run_min = jnp.full((TOK_BLK, 1), jnp.inf, dtype=jnp.float32)
    run_idx = jnp.zeros((TOK_BLK, 1), dtype=jnp.int32)
    for j in range(N_CODE_BLKS):
        eb = e[j * CODE_BLK:(j + 1) * CODE_BLK, :]
        se = jnp.sum(eb * eb, axis=1)
        mm = lax.dot_general(z, eb, (((1,), (1,)), ((), ())),
                             preferred_element_type=jnp.float32)
        dis = (sz + se[None, :]) - 2.0 * mm
        cmin = jnp.min(dis, axis=1, keepdims=True)
        iota = lax.broadcasted_iota(jnp.int32, dis.shape, 1)
        cidx = jnp.min(jnp.where(dis == cmin, iota, K_CODES),
                       axis=1, keepdims=True) + j * CODE_BLK
        upd = cmin < run_min
        run_min = jnp.where(upd, cmin, run_min)
        run_idx = jnp.where(upd, cidx, run_idx)

    idx_ref[...] = run_idx

    acc = jnp.zeros((TOK_BLK, LATENT), dtype=jnp.float32)
    for j in range(N_CODE_BLKS):
        eb = e[j * CODE_BLK:(j + 1) * CODE_BLK, :]
        iota = lax.broadcasted_iota(jnp.int32, (TOK_BLK, CODE_BLK), 1)
        oh = (run_idx == iota + j * CODE_BLK).astype(jnp.float32)
        acc += lax.dot_general(oh, eb, (((1,), (0,)), ((), ())),
                               preferred_element_type=jnp.float32)
    zq_ref[...] = acc

    @pl.when(i == 0)
    def _init():
        loss_ref[...] = jnp.zeros_like(loss_ref)

    loss_ref[...] += jnp.sum(run_min).reshape(1, 1)

    @pl.when(i == pl.num_programs(0) - 1)
    def _finalize():
        m = loss_ref[...] * (1.0 / (N_TOKENS * LATENT))
        loss_ref[...] = m + BETA * m


def _dist_argmin(z_flat, embedding_weight):
    grid = N_TOKENS // TOK_BLK
    idx2d, loss, zq = pl.pallas_call(
        _dist_argmin_body,
        grid=(grid,),
        in_specs=[
            pl.BlockSpec((TOK_BLK, LATENT), lambda i: (i, 0)),
            pl.BlockSpec((K_CODES, LATENT), lambda i: (0, 0)),
        ],
        out_specs=[
            pl.BlockSpec((TOK_BLK, 1), lambda i: (i, 0)),
            pl.BlockSpec((1, 1), lambda i: (0, 0)),
            pl.BlockSpec((TOK_BLK, LATENT), lambda i: (i, 0)),
        ],
        out_shape=[
            jax.ShapeDtypeStruct((N_TOKENS, 1), jnp.int32),
            jax.ShapeDtypeStruct((1, 1), jnp.float32),
            jax.ShapeDtypeStruct((N_TOKENS, LATENT), jnp.float32),
        ],
    )(z_flat, embedding_weight)
    return idx2d, loss, zq


def _gather_rows(table, idx2d):
    info = plsc.get_sparse_core_info()
    nw = info.num_cores * info.num_subcores
    rows_per_w = N_TOKENS // nw
    chunks = rows_per_w // 128

    @functools.partial(
        pl.kernel,
        mesh=plsc.VectorSubcoreMesh(core_axis_name="c", subcore_axis_name="s"),
        out_type=jax.ShapeDtypeStruct((N_TOKENS, 128), jnp.float32),
        scratch_types=[
            pltpu.VMEM((chunks, 128), jnp.int32),
            pltpu.VMEM((rows_per_w, 128), jnp.float32),
            pltpu.SemaphoreType.DMA,
        ],
    )
    def k(table_hbm, idx_hbm, out_hbm, idx_v, rows_v, sem):
        wid = lax.axis_index("s") * info.num_cores + lax.axis_index("c")
        base = wid * rows_per_w
        pltpu.sync_copy(idx_hbm.at[pl.ds(wid * chunks, chunks)], idx_v)
        cps = [
            pltpu.async_copy(table_hbm.at[idx_v.at[c]],
                             rows_v.at[pl.ds(c * 128, 128)], sem)
            for c in range(chunks)
        ]
        for cp in cps:
            cp.wait()
        pltpu.sync_copy(rows_v, out_hbm.at[pl.ds(base, rows_per_w)])

    return k(table, idx2d)


def kernel(z, embedding_weight):
    zp = jnp.transpose(z, (0, 2, 3, 1))
    z_flat = zp.reshape(-1, LATENT)

    idx2d, loss, zq_flat = _dist_argmin(z_flat, embedding_weight)
    idx = idx2d.reshape(-1)
    z_q = jnp.transpose(zq_flat.reshape(zp.shape), (0, 3, 1, 2))
    return (z_q, idx, loss.reshape(()))
